# baseline (speedup 1.0000x reference)
import functools

import jax
import jax.numpy as jnp
from jax import lax
from jax.experimental import pallas as pl
from jax.experimental.pallas import tpu as pltpu
from jax.experimental.pallas import tpu_sc as plsc

EPS = 1e-07
BN_EPS = 1e-05
NC = 2
NS = 16
L = 16
CH = 64


def _ea_body(attr_ref, wlo_ref, whi_ref, out_ref):
    a = attr_ref[...]
    out_ref[0] = jnp.dot(a, wlo_ref[...], preferred_element_type=jnp.float32)
    out_ref[1] = jnp.dot(a, whi_ref[...], preferred_element_type=jnp.float32)


def _edge_attr_tc(attr_p, Wlo, Whi):
    E_pad, DE = attr_p.shape
    H = Wlo.shape[1]
    BE = 4096
    while E_pad % BE:
        BE //= 2
    grid = (E_pad // BE,)
    return pl.pallas_call(
        _ea_body,
        grid=grid,
        in_specs=[
            pl.BlockSpec((BE, DE), lambda i: (i, 0)),
            pl.BlockSpec((DE, H), lambda i: (0, 0)),
            pl.BlockSpec((DE, H), lambda i: (0, 0)),
        ],
        out_specs=pl.BlockSpec((NC, BE, H), lambda i: (0, i, 0)),
        out_shape=jax.ShapeDtypeStruct((NC, E_pad, H), jnp.float32),
    )(attr_p, Wlo, Whi)


def _sc_edge_pass(x, srcp, dstp, ea2, zeros, N_pad, H, per_sub):
    D = x.shape[1]
    n_chunks = per_sub // CH
    rows_per_sub = N_pad // NS

    mesh = plsc.VectorSubcoreMesh(core_axis_name="c", subcore_axis_name="s")

    @functools.partial(
        pl.kernel,
        out_type=jax.ShapeDtypeStruct((NC, N_pad, D), jnp.float32),
        mesh=mesh,
        scratch_types=[
            pltpu.VMEM((CH,), jnp.int32),
            pltpu.VMEM((CH,), jnp.int32),
            pltpu.VMEM((CH,), jnp.int32),
            pltpu.VMEM((CH,), jnp.int32),
            pltpu.VMEM((CH,), jnp.int32),
            pltpu.VMEM((CH,), jnp.int32),
            pltpu.VMEM((CH,), jnp.int32),
            pltpu.VMEM((CH,), jnp.int32),
            pltpu.VMEM((2, CH, D), jnp.float32),
            pltpu.VMEM((2, CH, H), jnp.float32),
            pltpu.VMEM((2, CH, D), jnp.float32),
            pltpu.VMEM_SHARED((N_pad, D), jnp.float32),
        ] + [pltpu.SemaphoreType.DMA] * 10,
    )
    def sc_fn(x_hbm, src_hbm, dst_hbm, ea2_hbm, zero_hbm, o_hbm,
              sv0, sv1, sv2, sv3, dv0, dv1, dv2, dv3, xjv, eav, b, S,
              si0, si1, si2, si3, sg0, sg1, se0, se1, ss0, ss1):
        sv = (sv0, sv1, sv2, sv3)
        dv = (dv0, dv1, dv2, dv3)
        sem_i = (si0, si1, si2, si3)
        sem_g = (sg0, sg1)
        sem_e = (se0, se1)
        sem_s = (ss0, ss1)
        c = lax.axis_index("c")
        s = lax.axis_index("s")
        r0 = s * rows_per_sub
        hoff = c * H
        base0 = s * per_sub

        def issue_idx(j, m):
            base = base0 + j * CH
            pltpu.async_copy(src_hbm.at[pl.ds(base, CH)], sv[m], sem_i[m])
            pltpu.async_copy(dst_hbm.at[pl.ds(base, CH)], dv[m], sem_i[m])

        def wait_idx(j, m):
            base = base0 + j * CH
            pltpu.make_async_copy(
                src_hbm.at[pl.ds(base, CH)], sv[m], sem_i[m]).wait()
            pltpu.make_async_copy(
                dst_hbm.at[pl.ds(base, CH)], dv[m], sem_i[m]).wait()

        def issue_gather(m, p):
            pltpu.async_copy(x_hbm.at[sv[m]], xjv.at[p], sem_g[p])

        def wait_gather(m, p):
            pltpu.make_async_copy(
                x_hbm.at[sv[m]], xjv.at[p], sem_g[p]).wait()

        def ea_slice(j):
            return ea2_hbm.at[c, pl.ds(base0 + j * CH, CH)]

        def issue_ea(j, p):
            pltpu.async_copy(ea_slice(j), eav.at[p], sem_e[p])

        def wait_ea(j, p):
            pltpu.make_async_copy(ea_slice(j), eav.at[p], sem_e[p]).wait()

        def issue_scatter(m, p):
            pltpu.async_copy(b.at[p], S.at[dv[m]], sem_s[p], add=True)

        def wait_scatter(m, p):
            pltpu.make_async_copy(b.at[p], S.at[dv[m]], sem_s[p]).wait()

        def compute(p):
            @plsc.parallel_loop(0, CH, unroll=4)
            def edge_body(e):
                for g in range(H // L):
                    m = jnp.maximum(
                        xjv[p, e, pl.ds(hoff + g * L, L)]
                        + eav[p, e, pl.ds(g * L, L)], 0.0) + EPS
                    w = jnp.exp(m)
                    b[p, e, pl.ds(g * L, L)] = w
                    b[p, e, pl.ds(H + g * L, L)] = m * w

        def chunk_step(j, u, first=False, no_prev=False, no_idx=False,
                       no_next=False):
            p = u & 1
            q = p ^ 1
            if not first:
                wait_scatter((u + 2) % 4, p)
            if not no_next:
                wait_idx(j + 1, (u + 1) % 4)
                issue_gather((u + 1) % 4, q)
                issue_ea(j + 1, q)
            wait_gather(u, p)
            wait_ea(j, p)
            if not no_prev:
                issue_scatter((u + 3) % 4, q)
            compute(p)
            if not no_idx:
                issue_idx(j + 2, (u + 2) % 4)

        pltpu.sync_copy(zero_hbm.at[pl.ds(r0, rows_per_sub)],
                        S.at[pl.ds(r0, rows_per_sub)])
        plsc.subcore_barrier()

        issue_idx(0, 0)
        issue_idx(1, 1)
        wait_idx(0, 0)
        issue_gather(0, 0)
        issue_ea(0, 0)

        chunk_step(0, 0, first=True, no_prev=True)
        chunk_step(1, 1, first=True)
        chunk_step(2, 2)
        chunk_step(3, 3)

        def block_body(i4, carry):
            j = i4 * 4
            for u in range(4):
                chunk_step(j + u, u)
            return carry

        lax.fori_loop(1, n_chunks // 4 - 1, block_body, 0)

        jl = n_chunks - 4
        chunk_step(jl, 0)
        chunk_step(jl + 1, 1)
        chunk_step(jl + 2, 2, no_idx=True)
        chunk_step(jl + 3, 3, no_idx=True, no_next=True)
        issue_scatter(3, 1)
        wait_scatter(2, 0)
        wait_scatter(3, 1)

        plsc.subcore_barrier()
        pltpu.sync_copy(S.at[pl.ds(r0, rows_per_sub)],
                        o_hbm.at[c, pl.ds(r0, rows_per_sub)])

    return sc_fn(x, srcp, dstp, ea2, zeros)


def _mlp1_body(s_ref, x_ref, w1lo_ref, w1hi_ref, h_ref, stats_ref, *, half):
    i = pl.program_id(0)
    lo = s_ref[0]
    hi = s_ref[1]
    agg_lo = lo[:, half:] / (lo[:, :half] + 1e-16)
    agg_hi = hi[:, half:] / (hi[:, :half] + 1e-16)
    out_lo = agg_lo + x_ref[:, :half]
    out_hi = agg_hi + x_ref[:, half:]
    h = (jnp.dot(out_lo, w1lo_ref[...], preferred_element_type=jnp.float32)
         + jnp.dot(out_hi, w1hi_ref[...], preferred_element_type=jnp.float32))
    h_ref[...] = h
    part = jnp.concatenate(
        [jnp.sum(h, axis=0, keepdims=True),
         jnp.sum(h * h, axis=0, keepdims=True)], axis=0)

    @pl.when(i == 0)
    def _():
        stats_ref[...] = part

    @pl.when(i > 0)
    def _():
        stats_ref[...] += part


def _mlp1_tc(S, x, W1Tlo, W1Thi, N, H):
    D2 = W1Tlo.shape[1]
    D = x.shape[1]
    BN = 1000
    grid = (N // BN,)
    return pl.pallas_call(
        functools.partial(_mlp1_body, half=H),
        grid=grid,
        in_specs=[
            pl.BlockSpec((NC, BN, D), lambda i: (0, i, 0)),
            pl.BlockSpec((BN, D), lambda i: (i, 0)),
            pl.BlockSpec((H, D2), lambda i: (0, 0)),
            pl.BlockSpec((H, D2), lambda i: (0, 0)),
        ],
        out_specs=[
            pl.BlockSpec((BN, D2), lambda i: (i, 0)),
            pl.BlockSpec((2, D2), lambda i: (0, 0)),
        ],
        out_shape=[
            jax.ShapeDtypeStruct((N, D2), jnp.float32),
            jax.ShapeDtypeStruct((2, D2), jnp.float32),
        ],
    )(S, x, W1Tlo, W1Thi)


def _mlp2_body(h_ref, stats_ref, g_ref, b_ref, w2_ref, out_ref, *, n_rows):
    inv_n = 1.0 / n_rows
    mean = stats_ref[0:1] * inv_n
    msq = stats_ref[1:2] * inv_n
    var = msq - mean * mean
    inv = lax.rsqrt(var + BN_EPS)
    scale = g_ref[...] * inv
    shift = b_ref[...] - mean * scale
    hn = jnp.maximum(h_ref[...] * scale + shift, 0.0)
    out_ref[...] = jnp.dot(hn, w2_ref[...], preferred_element_type=jnp.float32)


def _mlp2_tc(h, stats, gamma2, beta2, W2T, N):
    D2, D = W2T.shape
    BN = 1000
    grid = (N // BN,)
    return pl.pallas_call(
        functools.partial(_mlp2_body, n_rows=float(N)),
        grid=grid,
        in_specs=[
            pl.BlockSpec((BN, D2), lambda i: (i, 0)),
            pl.BlockSpec((2, D2), lambda i: (0, 0)),
            pl.BlockSpec((1, D2), lambda i: (0, 0)),
            pl.BlockSpec((1, D2), lambda i: (0, 0)),
            pl.BlockSpec((D2, D), lambda i: (0, 0)),
        ],
        out_specs=pl.BlockSpec((BN, D), lambda i: (i, 0)),
        out_shape=jax.ShapeDtypeStruct((N, D), jnp.float32),
    )(h, stats, gamma2, beta2, W2T)


def kernel(x, edge_index, edge_attr, W_edge, W1, gamma, beta, W2):
    N, D = x.shape
    E = edge_index.shape[1]
    H = D // 2

    per_sub = -(-E // (NS * CH * 4)) * CH * 4
    E_pad = per_sub * NS
    N_pad = -(-N // (NS * 8)) * NS * 8

    src = edge_index[0]
    dst = edge_index[1]
    src_p = jnp.pad(src, (0, E_pad - E))
    dst_p = jnp.pad(dst, (0, E_pad - E), constant_values=N)
    attr_p = jnp.pad(edge_attr, ((0, E_pad - E), (0, 0)))
    zeros = jnp.zeros((N_pad, D), jnp.float32)

    WeT = W_edge.T
    ea2 = _edge_attr_tc(attr_p, WeT[:, :H], WeT[:, H:])

    S = _sc_edge_pass(x, src_p, dst_p, ea2, zeros, N_pad, H, per_sub)

    W1T = W1.T
    h, stats = _mlp1_tc(S, x, W1T[:H], W1T[H:], N, H)
    return _mlp2_tc(h, stats, gamma.reshape(1, -1), beta.reshape(1, -1),
                    W2.T, N)


# speedup vs baseline: 1.0355x; 1.0355x over previous
import functools

import jax
import jax.numpy as jnp
from jax import lax
from jax.experimental import pallas as pl
from jax.experimental.pallas import tpu as pltpu
from jax.experimental.pallas import tpu_sc as plsc

EPS = 1e-07
BN_EPS = 1e-05
NC = 2
NS = 16
L = 16
CH = 64


def _ea_body(attr_ref, wlo_ref, whi_ref, out_ref):
    a = attr_ref[...]
    out_ref[0] = jnp.dot(a, wlo_ref[...], preferred_element_type=jnp.float32)
    out_ref[1] = jnp.dot(a, whi_ref[...], preferred_element_type=jnp.float32)


def _edge_attr_tc(attr, E_pad, Wlo, Whi):
    DE = attr.shape[1]
    H = Wlo.shape[1]
    BE = 4096
    while E_pad % BE:
        BE //= 2
    grid = (E_pad // BE,)
    return pl.pallas_call(
        _ea_body,
        grid=grid,
        in_specs=[
            pl.BlockSpec((BE, DE), lambda i: (i, 0)),
            pl.BlockSpec((DE, H), lambda i: (0, 0)),
            pl.BlockSpec((DE, H), lambda i: (0, 0)),
        ],
        out_specs=pl.BlockSpec((NC, BE, H), lambda i: (0, i, 0)),
        out_shape=jax.ShapeDtypeStruct((NC, E_pad, H), jnp.float32),
    )(attr, Wlo, Whi)


def _sc_edge_pass(x, srcp, dstp, ea2, zeros, N_pad, H, per_sub):
    D = x.shape[1]
    n_chunks = per_sub // CH
    rows_per_sub = N_pad // NS

    mesh = plsc.VectorSubcoreMesh(core_axis_name="c", subcore_axis_name="s")

    @functools.partial(
        pl.kernel,
        out_type=jax.ShapeDtypeStruct((NC, N_pad, D), jnp.float32),
        mesh=mesh,
        scratch_types=[
            pltpu.VMEM((CH,), jnp.int32),
            pltpu.VMEM((CH,), jnp.int32),
            pltpu.VMEM((CH,), jnp.int32),
            pltpu.VMEM((CH,), jnp.int32),
            pltpu.VMEM((CH,), jnp.int32),
            pltpu.VMEM((CH,), jnp.int32),
            pltpu.VMEM((CH,), jnp.int32),
            pltpu.VMEM((CH,), jnp.int32),
            pltpu.VMEM((2, CH, D), jnp.float32),
            pltpu.VMEM((2, CH, H), jnp.float32),
            pltpu.VMEM((2, CH, D), jnp.float32),
            pltpu.VMEM_SHARED((N_pad, D), jnp.float32),
        ] + [pltpu.SemaphoreType.DMA] * 10,
    )
    def sc_fn(x_hbm, src_hbm, dst_hbm, ea2_hbm, zero_hbm, o_hbm,
              sv0, sv1, sv2, sv3, dv0, dv1, dv2, dv3, xjv, eav, b, S,
              si0, si1, si2, si3, sg0, sg1, se0, se1, ss0, ss1):
        sv = (sv0, sv1, sv2, sv3)
        dv = (dv0, dv1, dv2, dv3)
        sem_i = (si0, si1, si2, si3)
        sem_g = (sg0, sg1)
        sem_e = (se0, se1)
        sem_s = (ss0, ss1)
        c = lax.axis_index("c")
        s = lax.axis_index("s")
        r0 = s * rows_per_sub
        hoff = c * H
        base0 = s * per_sub

        def issue_idx(j, m):
            base = base0 + j * CH
            pltpu.async_copy(src_hbm.at[pl.ds(base, CH)], sv[m], sem_i[m])
            pltpu.async_copy(dst_hbm.at[pl.ds(base, CH)], dv[m], sem_i[m])

        def wait_idx(j, m):
            base = base0 + j * CH
            pltpu.make_async_copy(
                src_hbm.at[pl.ds(base, CH)], sv[m], sem_i[m]).wait()
            pltpu.make_async_copy(
                dst_hbm.at[pl.ds(base, CH)], dv[m], sem_i[m]).wait()

        def issue_gather(m, p):
            pltpu.async_copy(x_hbm.at[sv[m]], xjv.at[p], sem_g[p])

        def wait_gather(m, p):
            pltpu.make_async_copy(
                x_hbm.at[sv[m]], xjv.at[p], sem_g[p]).wait()

        def ea_slice(j):
            return ea2_hbm.at[c, pl.ds(base0 + j * CH, CH)]

        def issue_ea(j, p):
            pltpu.async_copy(ea_slice(j), eav.at[p], sem_e[p])

        def wait_ea(j, p):
            pltpu.make_async_copy(ea_slice(j), eav.at[p], sem_e[p]).wait()

        def issue_scatter(m, p):
            pltpu.async_copy(b.at[p], S.at[dv[m]], sem_s[p], add=True)

        def wait_scatter(m, p):
            pltpu.make_async_copy(b.at[p], S.at[dv[m]], sem_s[p]).wait()

        def compute(p):
            @plsc.parallel_loop(0, CH, unroll=4)
            def edge_body(e):
                for g in range(H // L):
                    m = jnp.maximum(
                        xjv[p, e, pl.ds(hoff + g * L, L)]
                        + eav[p, e, pl.ds(g * L, L)], 0.0) + EPS
                    w = jnp.exp(m)
                    b[p, e, pl.ds(g * L, L)] = w
                    b[p, e, pl.ds(H + g * L, L)] = m * w

        def chunk_step(j, u, first=False, no_prev=False, no_idx=False,
                       no_next=False):
            p = u & 1
            q = p ^ 1
            if not first:
                wait_scatter((u + 2) % 4, p)
            if not no_next:
                wait_idx(j + 1, (u + 1) % 4)
                issue_gather((u + 1) % 4, q)
                issue_ea(j + 1, q)
            wait_gather(u, p)
            wait_ea(j, p)
            if not no_prev:
                issue_scatter((u + 3) % 4, q)
            compute(p)
            if not no_idx:
                issue_idx(j + 2, (u + 2) % 4)

        pltpu.sync_copy(zero_hbm.at[pl.ds(r0, rows_per_sub)],
                        S.at[pl.ds(r0, rows_per_sub)])
        plsc.subcore_barrier()

        issue_idx(0, 0)
        issue_idx(1, 1)
        wait_idx(0, 0)
        issue_gather(0, 0)
        issue_ea(0, 0)

        chunk_step(0, 0, first=True, no_prev=True)
        chunk_step(1, 1, first=True)
        chunk_step(2, 2)
        chunk_step(3, 3)

        def block_body(i4, carry):
            j = i4 * 4
            for u in range(4):
                chunk_step(j + u, u)
            return carry

        lax.fori_loop(1, n_chunks // 4 - 1, block_body, 0)

        jl = n_chunks - 4
        chunk_step(jl, 0)
        chunk_step(jl + 1, 1)
        chunk_step(jl + 2, 2, no_idx=True)
        chunk_step(jl + 3, 3, no_idx=True, no_next=True)
        issue_scatter(3, 1)
        wait_scatter(2, 0)
        wait_scatter(3, 1)

        plsc.subcore_barrier()
        pltpu.sync_copy(S.at[pl.ds(r0, rows_per_sub)],
                        o_hbm.at[c, pl.ds(r0, rows_per_sub)])

    return sc_fn(x, srcp, dstp, ea2, zeros)


def _mlp1_body(s_ref, x_ref, w1lo_ref, w1hi_ref, h_ref, stats_ref, *, half):
    i = pl.program_id(0)
    lo = s_ref[0]
    hi = s_ref[1]
    agg_lo = lo[:, half:] / (lo[:, :half] + 1e-16)
    agg_hi = hi[:, half:] / (hi[:, :half] + 1e-16)
    out_lo = agg_lo + x_ref[:, :half]
    out_hi = agg_hi + x_ref[:, half:]
    h = (jnp.dot(out_lo, w1lo_ref[...], preferred_element_type=jnp.float32)
         + jnp.dot(out_hi, w1hi_ref[...], preferred_element_type=jnp.float32))
    h_ref[...] = h
    part = jnp.concatenate(
        [jnp.sum(h, axis=0, keepdims=True),
         jnp.sum(h * h, axis=0, keepdims=True)], axis=0)

    @pl.when(i == 0)
    def _():
        stats_ref[...] = part

    @pl.when(i > 0)
    def _():
        stats_ref[...] += part


def _mlp1_tc(S, x, W1Tlo, W1Thi, N, H):
    D2 = W1Tlo.shape[1]
    D = x.shape[1]
    BN = 1000
    grid = (N // BN,)
    return pl.pallas_call(
        functools.partial(_mlp1_body, half=H),
        grid=grid,
        in_specs=[
            pl.BlockSpec((NC, BN, D), lambda i: (0, i, 0)),
            pl.BlockSpec((BN, D), lambda i: (i, 0)),
            pl.BlockSpec((H, D2), lambda i: (0, 0)),
            pl.BlockSpec((H, D2), lambda i: (0, 0)),
        ],
        out_specs=[
            pl.BlockSpec((BN, D2), lambda i: (i, 0)),
            pl.BlockSpec((2, D2), lambda i: (0, 0)),
        ],
        out_shape=[
            jax.ShapeDtypeStruct((N, D2), jnp.float32),
            jax.ShapeDtypeStruct((2, D2), jnp.float32),
        ],
    )(S, x, W1Tlo, W1Thi)


def _mlp2_body(h_ref, stats_ref, g_ref, b_ref, w2_ref, out_ref, *, n_rows):
    inv_n = 1.0 / n_rows
    mean = stats_ref[0:1] * inv_n
    msq = stats_ref[1:2] * inv_n
    var = msq - mean * mean
    inv = lax.rsqrt(var + BN_EPS)
    scale = g_ref[...] * inv
    shift = b_ref[...] - mean * scale
    hn = jnp.maximum(h_ref[...] * scale + shift, 0.0)
    out_ref[...] = jnp.dot(hn, w2_ref[...], preferred_element_type=jnp.float32)


def _mlp2_tc(h, stats, gamma2, beta2, W2T, N):
    D2, D = W2T.shape
    BN = 1000
    grid = (N // BN,)
    return pl.pallas_call(
        functools.partial(_mlp2_body, n_rows=float(N)),
        grid=grid,
        in_specs=[
            pl.BlockSpec((BN, D2), lambda i: (i, 0)),
            pl.BlockSpec((2, D2), lambda i: (0, 0)),
            pl.BlockSpec((1, D2), lambda i: (0, 0)),
            pl.BlockSpec((1, D2), lambda i: (0, 0)),
            pl.BlockSpec((D2, D), lambda i: (0, 0)),
        ],
        out_specs=pl.BlockSpec((BN, D), lambda i: (i, 0)),
        out_shape=jax.ShapeDtypeStruct((N, D), jnp.float32),
    )(h, stats, gamma2, beta2, W2T)


def kernel(x, edge_index, edge_attr, W_edge, W1, gamma, beta, W2):
    N, D = x.shape
    E = edge_index.shape[1]
    H = D // 2

    per_sub = -(-E // (NS * CH * 4)) * CH * 4
    E_pad = per_sub * NS
    N_pad = -(-N // (NS * 8)) * NS * 8

    src = edge_index[0]
    dst = edge_index[1]
    src_p = jnp.pad(src, (0, E_pad - E))
    dst_p = jnp.pad(dst, (0, E_pad - E), constant_values=N)
    zeros = jnp.zeros((N_pad, D), jnp.float32)

    WeT = W_edge.T
    ea2 = _edge_attr_tc(edge_attr, E_pad, WeT[:, :H], WeT[:, H:])

    S = _sc_edge_pass(x, src_p, dst_p, ea2, zeros, N_pad, H, per_sub)

    W1T = W1.T
    h, stats = _mlp1_tc(S, x, W1T[:H], W1T[H:], N, H)
    return _mlp2_tc(h, stats, gamma.reshape(1, -1), beta.reshape(1, -1),
                    W2.T, N)
